# baseline (speedup 1.0000x reference)
import functools

import jax
import jax.numpy as jnp
from jax import lax
from jax.experimental import pallas as pl
from jax.experimental.pallas import tpu as pltpu
from jax.experimental.pallas import tpu_sc as plsc

N = 10000
E = 160000
F = 128
T_IN = 12
T_OUT = 4

NTILES = 32
RPT = 320
NPAD = NTILES * RPT
CH = 64
NCH = 92
EMAX = NCH * CH
G_TILE = EMAX // 16
GPC = CH // 16
SB_E = 256


def _mesh():
    return plsc.VectorSubcoreMesh(core_axis_name="c", subcore_axis_name="s")


def _make_prep():
    @functools.partial(
        pl.kernel,
        out_type=jax.ShapeDtypeStruct((2, T_IN, NTILES, EMAX), jnp.float32),
        mesh=_mesh(),
        compiler_params=pltpu.CompilerParams(needs_layout_passes=False),
        scratch_types=[
            pltpu.VMEM((EMAX,), jnp.int32),
            pltpu.VMEM((EMAX,), jnp.float32),
            pltpu.VMEM((EMAX,), jnp.float32),
            pltpu.VMEM((RPT,), jnp.float32),
            pltpu.VMEM((128,), jnp.float32),
        ],
    )
    def prep(rows_hbm, wraw_hbm, cnt_hbm, wn_hbm, rows_v, wv, wnv, deg_v, cnt_v):
        wid = lax.axis_index("s") * 2 + lax.axis_index("c")
        pltpu.sync_copy(cnt_hbm, cnt_v)
        iota16 = lax.iota(jnp.int32, 16)
        iota_gt = iota16 * G_TILE

        for d in range(2):
            pltpu.sync_copy(rows_hbm.at[d, wid], rows_v)
            cnt16 = plsc.load_gather(
                cnt_v,
                [jnp.full((16,), d * NTILES, jnp.int32)
                 + jnp.full((16,), wid, jnp.int32)]).astype(jnp.int32)

            def tbody(t, _, d=d, cnt16=cnt16):
                pltpu.sync_copy(wraw_hbm.at[d, t, wid], wv)

                def zb(i, _):
                    deg_v[pl.ds(i * 16, 16)] = jnp.zeros((16,), jnp.float32)
                    return 0
                lax.fori_loop(0, RPT // 16, zb, 0)

                def p1(g, _):
                    rows16 = rows_v[pl.ds(g * 16, 16)]
                    w16 = wv[pl.ds(g * 16, 16)]
                    msk = (iota_gt + jnp.full((16,), g, jnp.int32)) < cnt16
                    plsc.addupdate_scatter(deg_v, [rows16], w16, mask=msk)
                    return 0
                lax.fori_loop(0, G_TILE, p1, 0)

                def inv(i, _):
                    dd = deg_v[pl.ds(i * 16, 16)]
                    deg_v[pl.ds(i * 16, 16)] = jnp.where(
                        dd > 0, 1.0 / jnp.where(dd > 0, dd, 1.0), 0.0)
                    return 0
                lax.fori_loop(0, RPT // 16, inv, 0)

                def p2(g, _):
                    rows16 = rows_v[pl.ds(g * 16, 16)]
                    w16 = wv[pl.ds(g * 16, 16)]
                    wnv[pl.ds(g * 16, 16)] = w16 * plsc.load_gather(deg_v, [rows16])
                    return 0
                lax.fori_loop(0, G_TILE, p2, 0)
                pltpu.sync_copy(wnv, wn_hbm.at[d, t, wid])
                return 0

            lax.fori_loop(0, T_IN, tbody, 0)

    return prep


def _make_spmm(ncfg):
    @functools.partial(
        pl.kernel,
        out_type=jax.ShapeDtypeStruct((ncfg, NPAD * F), jnp.float32),
        mesh=_mesh(),
        compiler_params=pltpu.CompilerParams(needs_layout_passes=False),
        scratch_types=[
            pltpu.VMEM((NCH, CH), jnp.int32),
            pltpu.VMEM((RPT * F,), jnp.float32),
            pltpu.VMEM((2, CH, F), jnp.float32),
            pltpu.VMEM((2, CH, 16), jnp.int32),
            pltpu.VMEM((2, CH, 16), jnp.float32),
            pltpu.SemaphoreType.DMA,
            pltpu.SemaphoreType.DMA,
        ],
    )
    def spmm(tbl, init, didx_hbm, gidx_hbm, wx_hbm, out,
             gidx_v, acc, stg, didx_v, wx_v, sem0, sem1):
        wid = lax.axis_index("s") * 2 + lax.axis_index("c")
        sems = (sem0, sem1)

        def issue(cfg, d, c, b):
            pltpu.async_copy(tbl.at[cfg].at[gidx_v.at[c]], stg.at[b], sems[b])
            pltpu.async_copy(didx_hbm.at[d, wid, pl.ds(c * CH, CH)],
                             didx_v.at[b], sems[b])
            pltpu.async_copy(wx_hbm.at[d, wid, pl.ds(c * CH, CH)],
                             wx_v.at[b], sems[b])

        def drain(cfg, d, b):
            pltpu.make_async_copy(
                tbl.at[cfg].at[gidx_v.at[0]], stg.at[b], sems[b]).wait()
            pltpu.make_async_copy(
                didx_hbm.at[d, wid, pl.ds(0, CH)], didx_v.at[b], sems[b]).wait()
            pltpu.make_async_copy(
                wx_hbm.at[d, wid, pl.ds(0, CH)], wx_v.at[b], sems[b]).wait()

        def cbody(cfg, _):
            d = lax.rem(cfg, 2)
            pltpu.sync_copy(gidx_hbm.at[d, wid], gidx_v)
            pltpu.sync_copy(init.at[cfg, pl.ds(wid * RPT * F, RPT * F)], acc)
            issue(cfg, d, 0, 0)
            issue(cfg, d, 1, 1)

            def pbody(cp, _):
                for b in range(2):
                    c = cp * 2 + b
                    drain(cfg, d, b)
                    sb = stg.at[b]
                    db = didx_v.at[b]
                    wb = wx_v.at[b]

                    def ebody(e, _, sb=sb, db=db, wb=wb):
                        didx16 = db[e, pl.ds(0, 16)]
                        wx16 = wb[e, pl.ds(0, 16)]
                        for j in range(F // 16):
                            v = sb[e, pl.ds(j * 16, 16)]
                            plsc.addupdate_scatter(
                                acc, [didx16 + (j * 16)], v * wx16)
                        return 0
                    lax.fori_loop(0, CH, ebody, 0)

                    @pl.when(c + 2 < NCH)
                    def _(b=b, c=c):
                        issue(cfg, d, c + 2, b)
                return 0

            lax.fori_loop(0, NCH // 2, pbody, 0)
            pltpu.sync_copy(acc, out.at[cfg, pl.ds(wid * RPT * F, RPT * F)])
            return 0

        lax.fori_loop(0, ncfg, cbody, 0)

    return spmm


_BR = 512
_BR2 = 1024


def _mm_body(a_ref, h_ref, w_ref, bias_ref, o_ref):
    w = w_ref[...]
    o_ref[0] = (
        jnp.dot(a_ref[...], w[:F], preferred_element_type=jnp.float32)
        + jnp.dot(h_ref[...], w[F:], preferred_element_type=jnp.float32)
        + bias_ref[...]
    )


def _mm(a, h, wcat, bias, nout):
    return pl.pallas_call(
        _mm_body,
        grid=(NPAD // _BR, nout),
        in_specs=[
            pl.BlockSpec((_BR, F), lambda i, j: (i, 0)),
            pl.BlockSpec((_BR, F), lambda i, j: (i, 0)),
            pl.BlockSpec((2 * F, F), lambda i, j: (0, j)),
            pl.BlockSpec((1, F), lambda i, j: (0, j)),
        ],
        out_specs=pl.BlockSpec((1, _BR, F), lambda i, j: (j, i, 0)),
        out_shape=jax.ShapeDtypeStruct((nout, NPAD, F), jnp.float32),
    )(a, h, wcat, bias)


def _ew1_body(sb_ref, h_ref, z_ref, hr_ref):
    z = jax.nn.sigmoid(sb_ref[0] + sb_ref[1])
    r = jax.nn.sigmoid(sb_ref[2] + sb_ref[3])
    z_ref[...] = z
    hr_ref[...] = h_ref[...] * r


def _ew1(sb, h):
    return pl.pallas_call(
        _ew1_body,
        grid=(NPAD // _BR2,),
        in_specs=[
            pl.BlockSpec((4, _BR2, F), lambda i: (0, i, 0)),
            pl.BlockSpec((_BR2, F), lambda i: (i, 0)),
        ],
        out_specs=[
            pl.BlockSpec((_BR2, F), lambda i: (i, 0)),
            pl.BlockSpec((_BR2, F), lambda i: (i, 0)),
        ],
        out_shape=[
            jax.ShapeDtypeStruct((NPAD, F), jnp.float32),
            jax.ShapeDtypeStruct((NPAD, F), jnp.float32),
        ],
    )(sb, h)


def _ew2_body_pred(sb_ref, z_ref, h_ref, lw_ref, lb_ref, hn_ref, pr_ref):
    ht = jnp.tanh(sb_ref[0] + sb_ref[1])
    z = z_ref[...]
    hn = z * h_ref[...] + (1.0 - z) * ht
    hn_ref[...] = hn
    pr_ref[...] = (
        jnp.dot(hn, lw_ref[...], preferred_element_type=jnp.float32) + lb_ref[...]
    )


def _ew2_body(sb_ref, z_ref, h_ref, hn_ref):
    ht = jnp.tanh(sb_ref[0] + sb_ref[1])
    z = z_ref[...]
    hn_ref[...] = z * h_ref[...] + (1.0 - z) * ht


def _ew2(sb, z, h, lwb, lbb, with_pred):
    if not with_pred:
        return pl.pallas_call(
            _ew2_body,
            grid=(NPAD // _BR2,),
            in_specs=[
                pl.BlockSpec((2, _BR2, F), lambda i: (0, i, 0)),
                pl.BlockSpec((_BR2, F), lambda i: (i, 0)),
                pl.BlockSpec((_BR2, F), lambda i: (i, 0)),
            ],
            out_specs=pl.BlockSpec((_BR2, F), lambda i: (i, 0)),
            out_shape=jax.ShapeDtypeStruct((NPAD, F), jnp.float32),
        )(sb, z, h), None
    return pl.pallas_call(
        _ew2_body_pred,
        grid=(NPAD // _BR2,),
        in_specs=[
            pl.BlockSpec((2, _BR2, F), lambda i: (0, i, 0)),
            pl.BlockSpec((_BR2, F), lambda i: (i, 0)),
            pl.BlockSpec((_BR2, F), lambda i: (i, 0)),
            pl.BlockSpec((F, F), lambda i: (0, 0)),
            pl.BlockSpec((1, F), lambda i: (0, 0)),
        ],
        out_specs=[
            pl.BlockSpec((_BR2, F), lambda i: (i, 0)),
            pl.BlockSpec((_BR2, F), lambda i: (i, 0)),
        ],
        out_shape=[
            jax.ShapeDtypeStruct((NPAD, F), jnp.float32),
            jax.ShapeDtypeStruct((NPAD, F), jnp.float32),
        ],
    )(sb, z, h, lwb, lbb)


def _prep_side(scat, gath):
    perm = jnp.argsort(scat)
    rows_s = jnp.take(scat, perm)
    gath_s = jnp.take(gath, perm)
    bounds = jnp.arange(NTILES + 1, dtype=jnp.int32) * RPT
    estart = jnp.searchsorted(rows_s, bounds, side="left").astype(jnp.int32)
    cnt = estart[1:] - estart[:-1]
    j = jnp.arange(EMAX, dtype=jnp.int32)
    off = (j % 16) * G_TILE + j // 16
    pos = estart[:-1][:, None] + off[None, :]
    valid = off[None, :] < cnt[:, None]
    pos_c = jnp.clip(pos, 0, E - 1).astype(jnp.int32)
    pos_flat = pos_c.reshape(-1)
    rows_l = jnp.where(
        valid,
        jnp.take(rows_s, pos_flat).reshape(NTILES, EMAX)
        - (jnp.arange(NTILES, dtype=jnp.int32) * RPT)[:, None],
        0,
    ).astype(jnp.int32)
    gidx = jnp.where(
        valid, jnp.take(gath_s, pos_flat).reshape(NTILES, EMAX), 0
    ).astype(jnp.int32)
    return perm, pos_flat, valid, rows_l, gidx, cnt


def kernel(x_sequence, edge_index, edge_weight_sequence,
           W_z, b_z, W_r, b_r, W_h, b_h, lin_W, lin_b):
    src = edge_index[0].astype(jnp.int32)
    dst = edge_index[1].astype(jnp.int32)

    perm_f, pos_f, val_f, rows_f, gidx_f, cnt_f = _prep_side(src, dst)
    perm_b, pos_b, val_b, rows_b, gidx_b, cnt_b = _prep_side(dst, src)

    def _wpad(perm, pos_flat, valid):
        wp = edge_weight_sequence[:, perm]
        wp = wp[:, pos_flat].reshape(T_IN, NTILES, EMAX)
        return jnp.where(valid[None], wp, 0.0)

    rows_hbm = jnp.stack([rows_f, rows_b])
    gidx_hbm = jnp.stack([gidx_f, gidx_b]).reshape(2, NTILES, NCH, CH)
    wraw_hbm = jnp.stack([_wpad(perm_f, pos_f, val_f),
                          _wpad(perm_b, pos_b, val_b)])
    cnt_hbm = jnp.zeros((128,), jnp.float32).at[:64].set(
        jnp.stack([cnt_f, cnt_b]).reshape(64).astype(jnp.float32))

    wnorm = _make_prep()(rows_hbm, wraw_hbm, cnt_hbm)

    didx_hbm = (rows_hbm[..., None] * F
                + jnp.arange(16, dtype=jnp.int32))
    wx_all = jnp.broadcast_to(
        wnorm[:, :, :, :, None],
        (2, T_IN, NTILES, EMAX, 16))

    spmm4 = _make_spmm(4)
    spmm2 = _make_spmm(2)

    wcat_zr = jnp.concatenate(
        [W_z[0, 2], W_z[1, 2], W_r[0, 2], W_r[1, 2],
         W_z[0, 1], W_z[1, 1], W_r[0, 1], W_r[1, 1],
         W_z[0, 0] + W_z[1, 0], W_r[0, 0] + W_r[1, 0]], axis=1)
    bias_zr = jnp.concatenate(
        [jnp.zeros((8 * F,), jnp.float32), b_z, b_r]).reshape(1, 10 * F)
    wcat_h = jnp.concatenate(
        [W_h[0, 2], W_h[1, 2], W_h[0, 1], W_h[1, 1],
         W_h[0, 0] + W_h[1, 0]], axis=1)
    bias_h = jnp.concatenate(
        [jnp.zeros((4 * F,), jnp.float32), b_h]).reshape(1, 5 * F)
    lwb = jnp.broadcast_to(lin_W, (F, F))
    lbb = jnp.broadcast_to(lin_b.reshape(1, 1), (1, F))

    zeros_nf = jnp.zeros((NPAD, F), jnp.float32)

    xp = jnp.zeros((T_IN, NPAD, F), jnp.float32).at[:, :N].set(
        x_sequence.transpose(1, 0, 2))

    def spmm_wrap(op, tbls, inits, wx_t):
        outf = op(tbls, inits.reshape(inits.shape[0], NPAD * F),
                  didx_hbm, gidx_hbm, wx_t)
        return outf.reshape(outf.shape[0], NPAD, F)

    def cell(X, H, wx_t, with_pred):
        m = _mm(X, H, wcat_zr, bias_zr, 10)
        va = spmm_wrap(spmm4, m[0:4], m[4:8], wx_t)
        ib = jnp.stack([m[8], zeros_nf, m[9], zeros_nf])
        sb = spmm_wrap(spmm4, va, ib, wx_t)
        z, hr = _ew1(sb, H)
        mh = _mm(X, hr, wcat_h, bias_h, 5)
        vah = spmm_wrap(spmm2, mh[0:2], mh[2:4], wx_t)
        ibh = jnp.stack([mh[4], zeros_nf])
        sbh = spmm_wrap(spmm2, vah, ibh, wx_t)
        return _ew2(sbh, z, H, lwb, lbb, with_pred)

    h = zeros_nf
    for t in range(T_IN):
        h, _ = cell(xp[t], h, wx_all[:, t], False)
    wn_last = wx_all[:, T_IN - 1]
    preds = []
    for _ in range(T_OUT):
        h, pr = cell(h, h, wn_last, True)
        preds.append(pr[:N, :1])
    return jnp.stack(preds, axis=1)


# speedup vs baseline: 1.1636x; 1.1636x over previous
import functools

import jax
import jax.numpy as jnp
from jax import lax
from jax.experimental import pallas as pl
from jax.experimental.pallas import tpu as pltpu
from jax.experimental.pallas import tpu_sc as plsc

N = 10000
E = 160000
F = 128
T_IN = 12
T_OUT = 4

NTILES = 32
RPT = 320
NPAD = NTILES * RPT
CH = 48
NB = 8
NCH = 120
EMAX = NCH * CH
G_TILE = EMAX // 16
GPC = CH // 16
SB_E = 256


def _mesh():
    return plsc.VectorSubcoreMesh(core_axis_name="c", subcore_axis_name="s")


def _make_prep():
    @functools.partial(
        pl.kernel,
        out_type=jax.ShapeDtypeStruct((2, T_IN, NTILES, EMAX), jnp.float32),
        mesh=_mesh(),
        compiler_params=pltpu.CompilerParams(needs_layout_passes=False),
        scratch_types=[
            pltpu.VMEM((EMAX,), jnp.int32),
            pltpu.VMEM((EMAX,), jnp.float32),
            pltpu.VMEM((EMAX,), jnp.float32),
            pltpu.VMEM((RPT,), jnp.float32),
            pltpu.VMEM((128,), jnp.float32),
        ],
    )
    def prep(rows_hbm, wraw_hbm, cnt_hbm, wn_hbm, rows_v, wv, wnv, deg_v, cnt_v):
        wid = lax.axis_index("s") * 2 + lax.axis_index("c")
        pltpu.sync_copy(cnt_hbm, cnt_v)
        iota16 = lax.iota(jnp.int32, 16)
        iota_gt = iota16 * G_TILE

        for d in range(2):
            pltpu.sync_copy(rows_hbm.at[d, wid], rows_v)
            cnt16 = plsc.load_gather(
                cnt_v,
                [jnp.full((16,), d * NTILES, jnp.int32)
                 + jnp.full((16,), wid, jnp.int32)]).astype(jnp.int32)

            def tbody(t, _, d=d, cnt16=cnt16):
                pltpu.sync_copy(wraw_hbm.at[d, t, wid], wv)

                def zb(i, _):
                    deg_v[pl.ds(i * 16, 16)] = jnp.zeros((16,), jnp.float32)
                    return 0
                lax.fori_loop(0, RPT // 16, zb, 0)

                def p1(g, _):
                    rows16 = rows_v[pl.ds(g * 16, 16)]
                    w16 = wv[pl.ds(g * 16, 16)]
                    msk = (iota_gt + jnp.full((16,), g, jnp.int32)) < cnt16
                    plsc.addupdate_scatter(deg_v, [rows16], w16, mask=msk)
                    return 0
                lax.fori_loop(0, G_TILE, p1, 0)

                def inv(i, _):
                    dd = deg_v[pl.ds(i * 16, 16)]
                    deg_v[pl.ds(i * 16, 16)] = jnp.where(
                        dd > 0, 1.0 / jnp.where(dd > 0, dd, 1.0), 0.0)
                    return 0
                lax.fori_loop(0, RPT // 16, inv, 0)

                def p2(g, _):
                    rows16 = rows_v[pl.ds(g * 16, 16)]
                    w16 = wv[pl.ds(g * 16, 16)]
                    wnv[pl.ds(g * 16, 16)] = w16 * plsc.load_gather(deg_v, [rows16])
                    return 0
                lax.fori_loop(0, G_TILE, p2, 0)
                pltpu.sync_copy(wnv, wn_hbm.at[d, t, wid])
                return 0

            lax.fori_loop(0, T_IN, tbody, 0)

    return prep


def _make_spmm(ncfg):
    @functools.partial(
        pl.kernel,
        out_type=jax.ShapeDtypeStruct((ncfg, NPAD * F), jnp.float32),
        mesh=_mesh(),
        compiler_params=pltpu.CompilerParams(needs_layout_passes=False),
        scratch_types=[
            pltpu.VMEM((NCH, CH), jnp.int32),
            pltpu.VMEM((RPT * F,), jnp.float32),
            pltpu.VMEM((NB, CH, F), jnp.float32),
        ] + [pltpu.VMEM((CH * 16,), jnp.int32)] * NB
          + [pltpu.VMEM((CH * 16,), jnp.float32)] * NB
          + [pltpu.SemaphoreType.DMA] * NB,
    )
    def spmm(tbl, init, didx_hbm, gidx_hbm, wx_hbm, out,
             gidx_v, acc, stg, *rest):
        didx_v = rest[0:NB]
        wx_v = rest[NB:2 * NB]
        sems = rest[2 * NB:3 * NB]
        wid = lax.axis_index("s") * 2 + lax.axis_index("c")

        def issue(cfg, d, c, b):
            pltpu.async_copy(tbl.at[cfg].at[gidx_v.at[c]], stg.at[b], sems[b])
            pltpu.async_copy(didx_hbm.at[d, wid, c], didx_v[b], sems[b])
            pltpu.async_copy(wx_hbm.at[d, wid, c], wx_v[b], sems[b])

        def drain(cfg, d, b):
            pltpu.make_async_copy(
                tbl.at[cfg].at[gidx_v.at[0]], stg.at[b], sems[b]).wait()
            pltpu.make_async_copy(
                didx_hbm.at[d, wid, 0], didx_v[b], sems[b]).wait()
            pltpu.make_async_copy(
                wx_hbm.at[d, wid, 0], wx_v[b], sems[b]).wait()

        def cbody(cfg, _):
            d = lax.rem(cfg, 2)
            pltpu.sync_copy(gidx_hbm.at[d, wid], gidx_v)
            pltpu.sync_copy(init.at[cfg, pl.ds(wid * RPT * F, RPT * F)], acc)
            for b0 in range(NB):
                issue(cfg, d, b0, b0)

            def pbody(cp, _):
                for b in range(NB):
                    c = cp * NB + b
                    drain(cfg, d, b)
                    sb = stg.at[b]
                    db = didx_v[b]
                    wb = wx_v[b]

                    def ebody(e, _, sb=sb, db=db, wb=wb):
                        didx16 = db[pl.ds(e * 16, 16)]
                        wx16 = wb[pl.ds(e * 16, 16)]
                        for j in range(F // 16):
                            v = sb[e, pl.ds(j * 16, 16)]
                            plsc.addupdate_scatter(
                                acc, [didx16 + (j * 16)], v * wx16)
                        return 0
                    lax.fori_loop(0, CH, ebody, 0)

                    @pl.when(c + NB < NCH)
                    def _(b=b, c=c):
                        issue(cfg, d, c + NB, b)
                return 0

            lax.fori_loop(0, NCH // NB, pbody, 0)
            pltpu.sync_copy(acc, out.at[cfg, pl.ds(wid * RPT * F, RPT * F)])
            return 0

        lax.fori_loop(0, ncfg, cbody, 0)

    return spmm


_BR = 512
_BR2 = 1024


def _mm_body(a_ref, h_ref, w_ref, bias_ref, o_ref):
    w = w_ref[...]
    o_ref[0] = (
        jnp.dot(a_ref[...], w[:F], preferred_element_type=jnp.float32)
        + jnp.dot(h_ref[...], w[F:], preferred_element_type=jnp.float32)
        + bias_ref[...]
    )


def _mm(a, h, wcat, bias, nout):
    return pl.pallas_call(
        _mm_body,
        grid=(NPAD // _BR, nout),
        in_specs=[
            pl.BlockSpec((_BR, F), lambda i, j: (i, 0)),
            pl.BlockSpec((_BR, F), lambda i, j: (i, 0)),
            pl.BlockSpec((2 * F, F), lambda i, j: (0, j)),
            pl.BlockSpec((1, F), lambda i, j: (0, j)),
        ],
        out_specs=pl.BlockSpec((1, _BR, F), lambda i, j: (j, i, 0)),
        out_shape=jax.ShapeDtypeStruct((nout, NPAD, F), jnp.float32),
    )(a, h, wcat, bias)


def _ew1_body(sb_ref, h_ref, z_ref, hr_ref):
    z = jax.nn.sigmoid(sb_ref[0] + sb_ref[1])
    r = jax.nn.sigmoid(sb_ref[2] + sb_ref[3])
    z_ref[...] = z
    hr_ref[...] = h_ref[...] * r


def _ew1(sb, h):
    return pl.pallas_call(
        _ew1_body,
        grid=(NPAD // _BR2,),
        in_specs=[
            pl.BlockSpec((4, _BR2, F), lambda i: (0, i, 0)),
            pl.BlockSpec((_BR2, F), lambda i: (i, 0)),
        ],
        out_specs=[
            pl.BlockSpec((_BR2, F), lambda i: (i, 0)),
            pl.BlockSpec((_BR2, F), lambda i: (i, 0)),
        ],
        out_shape=[
            jax.ShapeDtypeStruct((NPAD, F), jnp.float32),
            jax.ShapeDtypeStruct((NPAD, F), jnp.float32),
        ],
    )(sb, h)


def _ew2_body_pred(sb_ref, z_ref, h_ref, lw_ref, lb_ref, hn_ref, pr_ref):
    ht = jnp.tanh(sb_ref[0] + sb_ref[1])
    z = z_ref[...]
    hn = z * h_ref[...] + (1.0 - z) * ht
    hn_ref[...] = hn
    pr_ref[...] = (
        jnp.dot(hn, lw_ref[...], preferred_element_type=jnp.float32) + lb_ref[...]
    )


def _ew2_body(sb_ref, z_ref, h_ref, hn_ref):
    ht = jnp.tanh(sb_ref[0] + sb_ref[1])
    z = z_ref[...]
    hn_ref[...] = z * h_ref[...] + (1.0 - z) * ht


def _ew2(sb, z, h, lwb, lbb, with_pred):
    if not with_pred:
        return pl.pallas_call(
            _ew2_body,
            grid=(NPAD // _BR2,),
            in_specs=[
                pl.BlockSpec((2, _BR2, F), lambda i: (0, i, 0)),
                pl.BlockSpec((_BR2, F), lambda i: (i, 0)),
                pl.BlockSpec((_BR2, F), lambda i: (i, 0)),
            ],
            out_specs=pl.BlockSpec((_BR2, F), lambda i: (i, 0)),
            out_shape=jax.ShapeDtypeStruct((NPAD, F), jnp.float32),
        )(sb, z, h), None
    return pl.pallas_call(
        _ew2_body_pred,
        grid=(NPAD // _BR2,),
        in_specs=[
            pl.BlockSpec((2, _BR2, F), lambda i: (0, i, 0)),
            pl.BlockSpec((_BR2, F), lambda i: (i, 0)),
            pl.BlockSpec((_BR2, F), lambda i: (i, 0)),
            pl.BlockSpec((F, F), lambda i: (0, 0)),
            pl.BlockSpec((1, F), lambda i: (0, 0)),
        ],
        out_specs=[
            pl.BlockSpec((_BR2, F), lambda i: (i, 0)),
            pl.BlockSpec((_BR2, F), lambda i: (i, 0)),
        ],
        out_shape=[
            jax.ShapeDtypeStruct((NPAD, F), jnp.float32),
            jax.ShapeDtypeStruct((NPAD, F), jnp.float32),
        ],
    )(sb, z, h, lwb, lbb)


def _prep_side(scat, gath):
    perm = jnp.argsort(scat)
    rows_s = jnp.take(scat, perm)
    gath_s = jnp.take(gath, perm)
    bounds = jnp.arange(NTILES + 1, dtype=jnp.int32) * RPT
    estart = jnp.searchsorted(rows_s, bounds, side="left").astype(jnp.int32)
    cnt = estart[1:] - estart[:-1]
    j = jnp.arange(EMAX, dtype=jnp.int32)
    off = (j % 16) * G_TILE + j // 16
    pos = estart[:-1][:, None] + off[None, :]
    valid = off[None, :] < cnt[:, None]
    pos_c = jnp.clip(pos, 0, E - 1).astype(jnp.int32)
    pos_flat = pos_c.reshape(-1)
    rows_l = jnp.where(
        valid,
        jnp.take(rows_s, pos_flat).reshape(NTILES, EMAX)
        - (jnp.arange(NTILES, dtype=jnp.int32) * RPT)[:, None],
        0,
    ).astype(jnp.int32)
    gidx = jnp.where(
        valid, jnp.take(gath_s, pos_flat).reshape(NTILES, EMAX), 0
    ).astype(jnp.int32)
    return perm, pos_flat, valid, rows_l, gidx, cnt


def kernel(x_sequence, edge_index, edge_weight_sequence,
           W_z, b_z, W_r, b_r, W_h, b_h, lin_W, lin_b):
    src = edge_index[0].astype(jnp.int32)
    dst = edge_index[1].astype(jnp.int32)

    perm_f, pos_f, val_f, rows_f, gidx_f, cnt_f = _prep_side(src, dst)
    perm_b, pos_b, val_b, rows_b, gidx_b, cnt_b = _prep_side(dst, src)

    def _wpad(perm, pos_flat, valid):
        wp = edge_weight_sequence[:, perm]
        wp = wp[:, pos_flat].reshape(T_IN, NTILES, EMAX)
        return jnp.where(valid[None], wp, 0.0)

    rows_hbm = jnp.stack([rows_f, rows_b])
    gidx_hbm = jnp.stack([gidx_f, gidx_b]).reshape(2, NTILES, NCH, CH)
    wraw_hbm = jnp.stack([_wpad(perm_f, pos_f, val_f),
                          _wpad(perm_b, pos_b, val_b)])
    cnt_hbm = jnp.zeros((128,), jnp.float32).at[:64].set(
        jnp.stack([cnt_f, cnt_b]).reshape(64).astype(jnp.float32))

    wnorm = _make_prep()(rows_hbm, wraw_hbm, cnt_hbm)

    didx_hbm = (rows_hbm[..., None] * F
                + jnp.arange(16, dtype=jnp.int32)).reshape(
                    2, NTILES, NCH, CH * 16)
    wx_all = jnp.broadcast_to(
        wnorm[:, :, :, :, None],
        (2, T_IN, NTILES, EMAX, 16)).reshape(2, T_IN, NTILES, NCH, CH * 16)

    spmm2 = _make_spmm(2)

    wcat_zr = jnp.concatenate(
        [W_z[0, 2], W_z[1, 2], W_r[0, 2], W_r[1, 2],
         W_z[0, 1], W_z[1, 1], W_r[0, 1], W_r[1, 1],
         W_z[0, 0] + W_z[1, 0], W_r[0, 0] + W_r[1, 0]], axis=1)
    bias_zr = jnp.concatenate(
        [jnp.zeros((8 * F,), jnp.float32), b_z, b_r]).reshape(1, 10 * F)
    wcat_h = jnp.concatenate(
        [W_h[0, 2], W_h[1, 2], W_h[0, 1], W_h[1, 1],
         W_h[0, 0] + W_h[1, 0]], axis=1)
    bias_h = jnp.concatenate(
        [jnp.zeros((4 * F,), jnp.float32), b_h]).reshape(1, 5 * F)
    lwb = jnp.broadcast_to(lin_W, (F, F))
    lbb = jnp.broadcast_to(lin_b.reshape(1, 1), (1, F))

    zeros_nf = jnp.zeros((NPAD, F), jnp.float32)

    xp = jnp.zeros((T_IN, NPAD, F), jnp.float32).at[:, :N].set(
        x_sequence.transpose(1, 0, 2))

    def spmm_wrap(op, tbls, inits, wx_t):
        outf = op(tbls, inits.reshape(inits.shape[0], NPAD * F),
                  didx_hbm, gidx_hbm, wx_t)
        return outf.reshape(outf.shape[0], NPAD, F)

    def cell(X, H, wx_t, with_pred):
        m = _mm(X, H, wcat_zr, bias_zr, 10)
        va_z = spmm_wrap(spmm2, m[0:2], m[4:6], wx_t)
        va_r = spmm_wrap(spmm2, m[2:4], m[6:8], wx_t)
        ib_z = jnp.stack([m[8], zeros_nf])
        ib_r = jnp.stack([m[9], zeros_nf])
        sb_z = spmm_wrap(spmm2, va_z, ib_z, wx_t)
        sb_r = spmm_wrap(spmm2, va_r, ib_r, wx_t)
        sb = jnp.concatenate([sb_z, sb_r], axis=0)
        z, hr = _ew1(sb, H)
        mh = _mm(X, hr, wcat_h, bias_h, 5)
        vah = spmm_wrap(spmm2, mh[0:2], mh[2:4], wx_t)
        ibh = jnp.stack([mh[4], zeros_nf])
        sbh = spmm_wrap(spmm2, vah, ibh, wx_t)
        return _ew2(sbh, z, H, lwb, lbb, with_pred)

    h = zeros_nf
    for t in range(T_IN):
        h, _ = cell(xp[t], h, wx_all[:, t], False)
    wn_last = wx_all[:, T_IN - 1]
    preds = []
    for _ in range(T_OUT):
        h, pr = cell(h, h, wn_last, True)
        preds.append(pr[:N, :1])
    return jnp.stack(preds, axis=1)
